# baseline (device time: 18080 ns/iter reference)
import jax
import jax.numpy as jnp
from jax import lax
from jax.experimental import pallas as pl
from jax.experimental.pallas import tpu as pltpu

N_DEV = 8
EPS = 1e-5


def kernel(x, t_emb, W_scale, W_shift):
    b, s, c = x.shape
    c_global = c * N_DEV

    def body(x_hbm, t_ref, ws_ref, wsh_ref, out_ref,
             xv, stats_ref, in_sem, send_sems, recv_sems):
        my_pos = lax.axis_index("i")

        barrier = pltpu.get_barrier_semaphore()
        for d in range(1, N_DEV):
            peer = (my_pos + d) % N_DEV
            pl.semaphore_signal(
                barrier, inc=1,
                device_id=(peer,), device_id_type=pl.DeviceIdType.MESH,
            )

        cp_in = pltpu.make_async_copy(x_hbm, xv, in_sem)
        cp_in.start()

        scale = jnp.dot(t_ref[...], ws_ref[...],
                        preferred_element_type=jnp.float32)
        shift = jnp.dot(t_ref[...], wsh_ref[...],
                        preferred_element_type=jnp.float32)
        sc1b = (1.0 + scale).astype(jnp.bfloat16)[:, None, :]
        shiftb = shift.astype(jnp.bfloat16)[:, None, :]

        cp_in.wait()
        xb = xv[...].astype(jnp.bfloat16)
        s1 = jnp.sum(xb, axis=-1, dtype=jnp.float32)
        s2 = jnp.sum(xb * xb, axis=-1, dtype=jnp.float32)
        local = jnp.concatenate([s1, s2], axis=0)
        stats_ref[pl.ds(my_pos, 1)] = local[None]

        pl.semaphore_wait(barrier, N_DEV - 1)

        sends = []
        for d in range(1, N_DEV):
            peer = (my_pos + d) % N_DEV
            rdma = pltpu.make_async_remote_copy(
                src_ref=stats_ref.at[my_pos],
                dst_ref=stats_ref.at[my_pos],
                send_sem=send_sems.at[d],
                recv_sem=recv_sems.at[my_pos],
                device_id=(peer,),
                device_id_type=pl.DeviceIdType.MESH,
            )
            rdma.start()
            sends.append(rdma)

        for d in range(1, N_DEV):
            peer = (my_pos + d) % N_DEV
            recv = pltpu.make_async_remote_copy(
                src_ref=stats_ref.at[peer],
                dst_ref=stats_ref.at[peer],
                send_sem=send_sems.at[d],
                recv_sem=recv_sems.at[peer],
                device_id=(peer,),
                device_id_type=pl.DeviceIdType.MESH,
            )
            recv.wait_recv()

        tot = jnp.sum(stats_ref[...], axis=0)
        mean = tot[0:b] / c_global
        var = tot[b:2 * b] / c_global - mean * mean
        inv = lax.rsqrt(var + EPS)
        minv = -mean * inv

        invb = inv.astype(jnp.bfloat16)[:, :, None]
        minvb = minv.astype(jnp.bfloat16)[:, :, None]
        out_ref[...] = (xb * invb + minvb) * sc1b + shiftb

        for rdma in sends:
            rdma.wait_send()

    return pl.pallas_call(
        body,
        out_shape=jax.ShapeDtypeStruct((b, s, c), jnp.bfloat16),
        in_specs=[
            pl.BlockSpec(memory_space=pl.ANY),
            pl.BlockSpec(memory_space=pltpu.VMEM),
            pl.BlockSpec(memory_space=pltpu.VMEM),
            pl.BlockSpec(memory_space=pltpu.VMEM),
        ],
        out_specs=pl.BlockSpec(memory_space=pltpu.VMEM),
        scratch_shapes=[
            pltpu.VMEM((b, s, c), jnp.float32),
            pltpu.VMEM((N_DEV, 2 * b, s), jnp.float32),
            pltpu.SemaphoreType.DMA,
            pltpu.SemaphoreType.DMA((N_DEV,)),
            pltpu.SemaphoreType.DMA((N_DEV,)),
        ],
        compiler_params=pltpu.CompilerParams(collective_id=0),
    )(x, t_emb, W_scale, W_shift)


# device time: 17373 ns/iter; 1.0407x vs baseline; 1.0407x over previous
import jax
import jax.numpy as jnp
from jax import lax
from jax.experimental import pallas as pl
from jax.experimental.pallas import tpu as pltpu

N_DEV = 8
EPS = 1e-5


def kernel(x, t_emb, W_scale, W_shift):
    b, s, c = x.shape
    c_global = c * N_DEV

    def body(x_ref, t_ref, ws_ref, wsh_ref, out_ref,
             stats_ref, send_sems, recv_sems):
        my_pos = lax.axis_index("i")

        barrier = pltpu.get_barrier_semaphore()
        for d in range(1, N_DEV):
            peer = (my_pos + d) % N_DEV
            pl.semaphore_signal(
                barrier, inc=1,
                device_id=(peer,), device_id_type=pl.DeviceIdType.MESH,
            )

        xb = x_ref[...].astype(jnp.bfloat16)
        s1 = jnp.sum(xb, axis=-1, dtype=jnp.float32)
        s2 = jnp.sum(xb * xb, axis=-1, dtype=jnp.float32)
        local = jnp.concatenate([s1, s2], axis=0)
        stats_ref[pl.ds(my_pos, 1)] = local[None]

        pl.semaphore_wait(barrier, N_DEV - 1)

        sends = []
        for d in range(1, N_DEV):
            peer = (my_pos + d) % N_DEV
            rdma = pltpu.make_async_remote_copy(
                src_ref=stats_ref.at[my_pos],
                dst_ref=stats_ref.at[my_pos],
                send_sem=send_sems.at[d],
                recv_sem=recv_sems.at[my_pos],
                device_id=(peer,),
                device_id_type=pl.DeviceIdType.MESH,
            )
            rdma.start()
            sends.append(rdma)

        scale = jnp.dot(t_ref[...], ws_ref[...],
                        preferred_element_type=jnp.float32)
        shift = jnp.dot(t_ref[...], wsh_ref[...],
                        preferred_element_type=jnp.float32)
        sc1 = 1.0 + scale

        for d in range(1, N_DEV):
            peer = (my_pos + d) % N_DEV
            recv = pltpu.make_async_remote_copy(
                src_ref=stats_ref.at[peer],
                dst_ref=stats_ref.at[peer],
                send_sem=send_sems.at[d],
                recv_sem=recv_sems.at[peer],
                device_id=(peer,),
                device_id_type=pl.DeviceIdType.MESH,
            )
            recv.wait_recv()

        tot = jnp.sum(stats_ref[...], axis=0)
        mean = tot[0:b] / c_global
        var = tot[b:2 * b] / c_global - mean * mean
        inv = lax.rsqrt(var + EPS)
        minv = -mean * inv

        invb = inv.astype(jnp.bfloat16)[:, :, None]
        minvb = minv.astype(jnp.bfloat16)[:, :, None]
        sc1b = sc1.astype(jnp.bfloat16)[:, None, :]
        shiftb = shift.astype(jnp.bfloat16)[:, None, :]
        out_ref[...] = (xb * invb + minvb) * sc1b + shiftb

        for rdma in sends:
            rdma.wait_send()

    return pl.pallas_call(
        body,
        out_shape=jax.ShapeDtypeStruct((b, s, c), jnp.bfloat16),
        in_specs=[
            pl.BlockSpec(memory_space=pltpu.VMEM),
            pl.BlockSpec(memory_space=pltpu.VMEM),
            pl.BlockSpec(memory_space=pltpu.VMEM),
            pl.BlockSpec(memory_space=pltpu.VMEM),
        ],
        out_specs=pl.BlockSpec(memory_space=pltpu.VMEM),
        scratch_shapes=[
            pltpu.VMEM((N_DEV, 2 * b, s), jnp.float32),
            pltpu.SemaphoreType.DMA((N_DEV,)),
            pltpu.SemaphoreType.DMA((N_DEV,)),
        ],
        compiler_params=pltpu.CompilerParams(collective_id=0),
    )(x, t_emb, W_scale, W_shift)
